# baseline (speedup 1.0000x reference)
import functools

import jax
import jax.numpy as jnp
from jax.experimental import pallas as pl
from jax.experimental.pallas import tpu as pltpu

X_GRID = 512
Y_GRID = 512


def _scatter_body(idx_ref, feat_ref, canvas_ref, out_ref):
    del idx_ref, canvas_ref
    out_ref[...] = feat_ref[...]


def _conv_body(canvas_ref, w_ref, b_ref, out_ref):
    x = canvas_ref[...]
    w = w_ref[...]
    acc = jax.lax.dot_general(
        w, x, (((1,), (1,)), ((), ())), preferred_element_type=jnp.float32
    )
    out_ref[...] = acc + b_ref[...]


@jax.jit
def _run(pillar_features, voxel_indices, conv_w, conv_b):
    V, C_in = pillar_features.shape
    C_out = conv_w.shape[0]
    N = X_GRID * Y_GRID

    canvas0 = jnp.zeros((N, 1, C_in), jnp.float32)
    feats3 = pillar_features.reshape(V, 1, C_in)
    flat_idx = voxel_indices[:, 0] * Y_GRID + voxel_indices[:, 1]

    scatter = pl.pallas_call(
        _scatter_body,
        grid_spec=pltpu.PrefetchScalarGridSpec(
            num_scalar_prefetch=1,
            grid=(V,),
            in_specs=[
                pl.BlockSpec((1, 1, C_in), lambda v, idx: (v, 0, 0)),
                pl.BlockSpec((1, 1, C_in), lambda v, idx: (0, 0, 0)),
            ],
            out_specs=pl.BlockSpec(
                (1, 1, C_in),
                lambda v, idx: (idx[v], 0, 0),
            ),
        ),
        out_shape=jax.ShapeDtypeStruct((N, 1, C_in), jnp.float32),
        input_output_aliases={2: 0},
    )
    canvas = scatter(flat_idx, feats3, canvas0)

    TILE = 4096
    conv = pl.pallas_call(
        _conv_body,
        grid=(N // TILE,),
        in_specs=[
            pl.BlockSpec((TILE, C_in), lambda t: (t, 0)),
            pl.BlockSpec((C_out, C_in), lambda t: (0, 0)),
            pl.BlockSpec((C_out, 1), lambda t: (0, 0)),
        ],
        out_specs=pl.BlockSpec((C_out, TILE), lambda t: (0, t)),
        out_shape=jax.ShapeDtypeStruct((C_out, N), jnp.float32),
    )
    out = conv(canvas.reshape(N, C_in), conv_w, conv_b.reshape(C_out, 1))
    return out.reshape(1, C_out, X_GRID, Y_GRID)


def kernel(pillar_features, voxel_indices, batch_size, conv_w, conv_b):
    del batch_size
    return _run(pillar_features, voxel_indices, conv_w, conv_b)


# speedup vs baseline: 32.3907x; 32.3907x over previous
import functools

import jax
import jax.numpy as jnp
from jax import lax
from jax.experimental import pallas as pl
from jax.experimental.pallas import tpu as pltpu
from jax.experimental.pallas import tpu_sc as plsc

X_GRID = 512
Y_GRID = 512
N_CELLS = X_GRID * Y_GRID
V_PILLARS = 20000
C_IN = 64
C_OUT = 64

NUM_WORKERS = 32
CELLS_PER_W = N_CELLS // NUM_WORKERS
CHUNK = 640
V_PAD = NUM_WORKERS * CHUNK
IDX_ROWS = CHUNK // 128
LANES = 16
CONV_TILE = 4096
DUMP_ROW = N_CELLS


def _winner_map_body(flat_hbm, map_hbm, flat_v, loc_v, id_v, map_v, sem):
    nc = plsc.get_sparse_core_info().num_cores
    wid = lax.axis_index("s") * nc + lax.axis_index("c")
    base_c = wid * CELLS_PER_W

    pltpu.async_copy(flat_hbm, flat_v, sem).wait()

    def init_body(i, _):
        map_v[pl.ds(i * LANES, LANES)] = jnp.full((LANES,), -1, jnp.int32)
        return 0

    lax.fori_loop(0, CELLS_PER_W // LANES, init_body, 0)

    lane_iota = lax.iota(jnp.int32, LANES)

    def scan_body(i, cnt):
        f = flat_v[pl.ds(i * LANES, LANES)]
        loc = f - base_c
        m = (f >= base_c) & (f < base_c + CELLS_PER_W)
        ids = lane_iota + i * LANES
        mi = m.astype(jnp.int32)
        pos = cnt + plsc.cumsum(mi) - mi
        plsc.store_scatter(loc_v, [pos], loc, mask=m)
        plsc.store_scatter(id_v, [pos], ids, mask=m)
        return cnt + plsc.all_reduce_population_count(m)[0]

    cnt = lax.fori_loop(0, V_PILLARS // LANES, scan_body, 0)

    lane0 = lane_iota == 0

    def replay_body(i, _):
        loc = loc_v[pl.ds(i, LANES)]
        ids = id_v[pl.ds(i, LANES)]
        plsc.store_scatter(map_v, [loc], ids, mask=lane0)
        return 0

    lax.fori_loop(0, cnt, replay_body, 0)

    pltpu.async_copy(map_v, map_hbm.at[pl.ds(base_c, CELLS_PER_W)], sem).wait()


def _dedup_scatter_body(
    flat_hbm, feats_hbm, map_hbm, canvas_hbm, idx_v, mapv_v, sidx_v, feats_v, sem
):
    nc = plsc.get_sparse_core_info().num_cores
    wid = lax.axis_index("s") * nc + lax.axis_index("c")
    base = wid * CHUNK

    for j in range(IDX_ROWS):
        pltpu.async_copy(
            flat_hbm.at[pl.ds(base + j * 128, 128)], idx_v.at[j], sem
        ).wait()
    for j in range(IDX_ROWS):
        pltpu.async_copy(
            map_hbm.at[idx_v.at[j]], mapv_v.at[pl.ds(j * 128, 128)], sem
        ).wait()

    lane_iota = lax.iota(jnp.int32, LANES)
    for k in range(CHUNK // LANES):
        j, o = divmod(k * LANES, 128)
        mv = mapv_v[pl.ds(k * LANES, LANES)]
        fv = idx_v[j, pl.ds(o, LANES)]
        ids = lane_iota + (base + k * LANES)
        win = mv == ids
        sidx_v[j, pl.ds(o, LANES)] = jnp.where(
            win, fv, jnp.full((LANES,), DUMP_ROW, jnp.int32)
        )

    n_real = V_PILLARS - (NUM_WORKERS - 1) * CHUNK

    @pl.when(wid < NUM_WORKERS - 1)
    def _copy_full():
        pltpu.async_copy(feats_hbm.at[pl.ds(base, CHUNK)], feats_v, sem).wait()

    @pl.when(wid == NUM_WORKERS - 1)
    def _copy_tail():
        pltpu.async_copy(
            feats_hbm.at[pl.ds(base, n_real)],
            feats_v.at[pl.ds(0, n_real)],
            sem,
        ).wait()

    for j in range(IDX_ROWS):
        pltpu.async_copy(
            feats_v.at[pl.ds(j * 128, 128)],
            canvas_hbm.at[sidx_v.at[j]],
            sem,
        ).wait()


def _conv_body(canvas_ref, map_ref, w_ref, b_ref, out_ref):
    x = canvas_ref[:, :C_IN]
    acc = lax.dot_general(
        w_ref[...], x, (((1,), (1,)), ((), ())), preferred_element_type=jnp.float32
    )
    occupied = map_ref[0, 0, :] >= 0
    out_ref[...] = jnp.where(occupied[None, :], acc, 0.0) + b_ref[...]


@jax.jit
def _run(pillar_features, voxel_indices, conv_w, conv_b):
    flat = voxel_indices[:, 0] * Y_GRID + voxel_indices[:, 1]
    flat_pad = jnp.concatenate(
        [flat, jnp.zeros((V_PAD - V_PILLARS,), jnp.int32)]
    )
    feats_pad = jnp.concatenate(
        [pillar_features, jnp.zeros((V_PILLARS, 128 - C_IN), jnp.float32)], axis=1
    )

    mesh = plsc.VectorSubcoreMesh(core_axis_name="c", subcore_axis_name="s")

    winner_map = pl.kernel(
        _winner_map_body,
        out_type=jax.ShapeDtypeStruct((N_CELLS,), jnp.int32),
        compiler_params=pltpu.CompilerParams(needs_layout_passes=False),
        mesh=mesh,
        scratch_types=[
            pltpu.VMEM((V_PILLARS,), jnp.int32),
            pltpu.VMEM((V_PAD,), jnp.int32),
            pltpu.VMEM((V_PAD,), jnp.int32),
            pltpu.VMEM((CELLS_PER_W,), jnp.int32),
            pltpu.SemaphoreType.DMA,
        ],
    )
    cell_map = winner_map(flat)

    scatter = pl.kernel(
        _dedup_scatter_body,
        out_type=jax.ShapeDtypeStruct((N_CELLS + CONV_TILE, 128), jnp.float32),
        compiler_params=pltpu.CompilerParams(needs_layout_passes=False),
        mesh=mesh,
        scratch_types=[
            pltpu.VMEM((IDX_ROWS, 128), jnp.int32),
            pltpu.VMEM((CHUNK,), jnp.int32),
            pltpu.VMEM((IDX_ROWS, 128), jnp.int32),
            pltpu.VMEM((CHUNK, 128), jnp.float32),
            pltpu.SemaphoreType.DMA,
        ],
    )
    canvas = scatter(flat_pad, feats_pad, cell_map)

    conv = pl.pallas_call(
        _conv_body,
        grid=(N_CELLS // CONV_TILE,),
        in_specs=[
            pl.BlockSpec((CONV_TILE, 128), lambda t: (t, 0)),
            pl.BlockSpec((1, 1, CONV_TILE), lambda t: (t, 0, 0)),
            pl.BlockSpec((C_OUT, C_IN), lambda t: (0, 0)),
            pl.BlockSpec((C_OUT, 1), lambda t: (0, 0)),
        ],
        out_specs=pl.BlockSpec((C_OUT, CONV_TILE), lambda t: (0, t)),
        out_shape=jax.ShapeDtypeStruct((C_OUT, N_CELLS), jnp.float32),
    )
    out = conv(
        canvas,
        cell_map.reshape(N_CELLS // CONV_TILE, 1, CONV_TILE),
        conv_w,
        conv_b.reshape(C_OUT, 1),
    )
    return out.reshape(1, C_OUT, X_GRID, Y_GRID)


def kernel(pillar_features, voxel_indices, batch_size, conv_w, conv_b):
    del batch_size
    return _run(pillar_features, voxel_indices, conv_w, conv_b)


# speedup vs baseline: 33.1365x; 1.0230x over previous
import functools

import jax
import jax.numpy as jnp
from jax import lax
from jax.experimental import pallas as pl
from jax.experimental.pallas import tpu as pltpu
from jax.experimental.pallas import tpu_sc as plsc

X_GRID = 512
Y_GRID = 512
N_CELLS = X_GRID * Y_GRID
V_PILLARS = 20000
C_IN = 64
C_OUT = 64

NUM_WORKERS = 32
CELLS_PER_W = N_CELLS // NUM_WORKERS
CHUNK = 640
V_PAD = NUM_WORKERS * CHUNK
IDX_ROWS = CHUNK // 128
LANES = 16
CONV_TILE = 4096
DUMP_ROW = N_CELLS


def _winner_map_body(flat_hbm, map_hbm, flat_v, loc_v, id_v, map_v, sem):
    nc = plsc.get_sparse_core_info().num_cores
    wid = lax.axis_index("s") * nc + lax.axis_index("c")
    base_c = wid * CELLS_PER_W

    pltpu.async_copy(flat_hbm, flat_v, sem).wait()

    def init_body(i, _):
        map_v[pl.ds(i * LANES, LANES)] = jnp.full((LANES,), -1, jnp.int32)
        return 0

    lax.fori_loop(0, CELLS_PER_W // LANES, init_body, 0)

    lane_iota = lax.iota(jnp.int32, LANES)

    def scan_body(i, cnt):
        f = flat_v[pl.ds(i * LANES, LANES)]
        loc = f - base_c
        m = (f >= base_c) & (f < base_c + CELLS_PER_W)
        ids = lane_iota + i * LANES
        mi = m.astype(jnp.int32)
        pos = cnt + plsc.cumsum(mi) - mi
        plsc.store_scatter(loc_v, [pos], loc, mask=m)
        plsc.store_scatter(id_v, [pos], ids, mask=m)
        return cnt + plsc.all_reduce_population_count(m)[0]

    cnt = lax.fori_loop(0, V_PILLARS // LANES, scan_body, 0)

    lane0 = lane_iota == 0

    def replay_body(i, _):
        loc = loc_v[pl.ds(i, LANES)]
        ids = id_v[pl.ds(i, LANES)]
        plsc.store_scatter(map_v, [loc], ids, mask=lane0)
        return 0

    lax.fori_loop(0, cnt, replay_body, 0)

    pltpu.async_copy(map_v, map_hbm.at[pl.ds(base_c, CELLS_PER_W)], sem).wait()


def _dedup_scatter_body(
    flat_hbm, feats_hbm, map_hbm, canvas_hbm, idx_v, mapv_v, sidx_v, feats_v,
    sem, sem2
):
    nc = plsc.get_sparse_core_info().num_cores
    wid = lax.axis_index("s") * nc + lax.axis_index("c")
    base = wid * CHUNK

    n_real = V_PILLARS - (NUM_WORKERS - 1) * CHUNK

    @pl.when(wid < NUM_WORKERS - 1)
    def _copy_full():
        pltpu.async_copy(feats_hbm.at[pl.ds(base, CHUNK)], feats_v, sem2)

    @pl.when(wid == NUM_WORKERS - 1)
    def _copy_tail():
        pltpu.async_copy(
            feats_hbm.at[pl.ds(base, n_real)],
            feats_v.at[pl.ds(0, n_real)],
            sem2,
        )

    flat_cps = [
        pltpu.async_copy(flat_hbm.at[pl.ds(base + j * 128, 128)], idx_v.at[j], sem)
        for j in range(IDX_ROWS)
    ]
    for c in flat_cps:
        c.wait()
    gather_cps = [
        pltpu.async_copy(map_hbm.at[idx_v.at[j]], mapv_v.at[pl.ds(j * 128, 128)], sem)
        for j in range(IDX_ROWS)
    ]
    for c in gather_cps:
        c.wait()

    lane_iota = lax.iota(jnp.int32, LANES)
    for k in range(CHUNK // LANES):
        j, o = divmod(k * LANES, 128)
        mv = mapv_v[pl.ds(k * LANES, LANES)]
        fv = idx_v[j, pl.ds(o, LANES)]
        ids = lane_iota + (base + k * LANES)
        win = mv == ids
        sidx_v[j, pl.ds(o, LANES)] = jnp.where(
            win, fv, jnp.full((LANES,), DUMP_ROW, jnp.int32)
        )

    @pl.when(wid < NUM_WORKERS - 1)
    def _wait_full():
        pltpu.make_async_copy(feats_hbm.at[pl.ds(base, CHUNK)], feats_v, sem2).wait()

    @pl.when(wid == NUM_WORKERS - 1)
    def _wait_tail():
        pltpu.make_async_copy(
            feats_hbm.at[pl.ds(base, n_real)],
            feats_v.at[pl.ds(0, n_real)],
            sem2,
        ).wait()

    scatter_cps = [
        pltpu.async_copy(
            feats_v.at[pl.ds(j * 128, 128)],
            canvas_hbm.at[sidx_v.at[j]],
            sem,
        )
        for j in range(IDX_ROWS)
    ]
    for c in scatter_cps:
        c.wait()


def _pad_body(in_ref, out_ref):
    x = in_ref[...]
    out_ref[...] = jnp.concatenate(
        [x, jnp.zeros((x.shape[0], 128 - C_IN), jnp.float32)], axis=1
    )


def _conv_body(canvas_ref, map_ref, w_ref, b_ref, out_ref):
    x = canvas_ref[:, :C_IN]
    acc = lax.dot_general(
        w_ref[...], x, (((1,), (1,)), ((), ())), preferred_element_type=jnp.float32
    )
    occupied = map_ref[0, 0, :] >= 0
    out_ref[...] = jnp.where(occupied[None, :], acc, 0.0) + b_ref[...]


@jax.jit
def _run(pillar_features, voxel_indices, conv_w, conv_b):
    flat = voxel_indices[:, 0] * Y_GRID + voxel_indices[:, 1]
    flat_pad = jnp.concatenate(
        [flat, jnp.zeros((V_PAD - V_PILLARS,), jnp.int32)]
    )
    PAD_TILE = 2000
    pad = pl.pallas_call(
        _pad_body,
        grid=(V_PILLARS // PAD_TILE,),
        in_specs=[pl.BlockSpec((PAD_TILE, C_IN), lambda t: (t, 0))],
        out_specs=pl.BlockSpec((PAD_TILE, 128), lambda t: (t, 0)),
        out_shape=jax.ShapeDtypeStruct((V_PILLARS, 128), jnp.float32),
    )
    feats_pad = pad(pillar_features)

    mesh = plsc.VectorSubcoreMesh(core_axis_name="c", subcore_axis_name="s")

    winner_map = pl.kernel(
        _winner_map_body,
        out_type=jax.ShapeDtypeStruct((N_CELLS,), jnp.int32),
        compiler_params=pltpu.CompilerParams(needs_layout_passes=False),
        mesh=mesh,
        scratch_types=[
            pltpu.VMEM((V_PILLARS,), jnp.int32),
            pltpu.VMEM((V_PAD,), jnp.int32),
            pltpu.VMEM((V_PAD,), jnp.int32),
            pltpu.VMEM((CELLS_PER_W,), jnp.int32),
            pltpu.SemaphoreType.DMA,
        ],
    )
    cell_map = winner_map(flat)

    scatter = pl.kernel(
        _dedup_scatter_body,
        out_type=jax.ShapeDtypeStruct((N_CELLS + CONV_TILE, 128), jnp.float32),
        compiler_params=pltpu.CompilerParams(needs_layout_passes=False),
        mesh=mesh,
        scratch_types=[
            pltpu.VMEM((IDX_ROWS, 128), jnp.int32),
            pltpu.VMEM((CHUNK,), jnp.int32),
            pltpu.VMEM((IDX_ROWS, 128), jnp.int32),
            pltpu.VMEM((CHUNK, 128), jnp.float32),
            pltpu.SemaphoreType.DMA,
            pltpu.SemaphoreType.DMA,
        ],
    )
    canvas = scatter(flat_pad, feats_pad, cell_map)

    conv = pl.pallas_call(
        _conv_body,
        grid=(N_CELLS // CONV_TILE,),
        in_specs=[
            pl.BlockSpec((CONV_TILE, 128), lambda t: (t, 0)),
            pl.BlockSpec((1, 1, CONV_TILE), lambda t: (t, 0, 0)),
            pl.BlockSpec((C_OUT, C_IN), lambda t: (0, 0)),
            pl.BlockSpec((C_OUT, 1), lambda t: (0, 0)),
        ],
        out_specs=pl.BlockSpec((C_OUT, CONV_TILE), lambda t: (0, t)),
        out_shape=jax.ShapeDtypeStruct((C_OUT, N_CELLS), jnp.float32),
    )
    out = conv(
        canvas,
        cell_map.reshape(N_CELLS // CONV_TILE, 1, CONV_TILE),
        conv_w,
        conv_b.reshape(C_OUT, 1),
    )
    return out.reshape(1, C_OUT, X_GRID, Y_GRID)


def kernel(pillar_features, voxel_indices, batch_size, conv_w, conv_b):
    del batch_size
    return _run(pillar_features, voxel_indices, conv_w, conv_b)
